# Initial kernel scaffold; baseline (speedup 1.0000x reference)
import functools

import jax
import jax.numpy as jnp
from jax import lax
from jax.experimental import pallas as pl
from jax.experimental.pallas import tpu as pltpu
from jax.experimental.pallas import tpu_sc as plsc

N = 10000
E = 320000
F = 32

NC = 2
NS = 16
NW = NC * NS
CHUNK = 128
CHUNKS = 80
E_PAD = NW * CHUNKS * CHUNK
N_ACC = 10016
SLAB = N_ACC // NS


def _seg_body(feat, srcs, dsts, zeros, out, src_v, dst_v, rows_a, rows_b, acc):
    c = lax.axis_index("c")
    s = lax.axis_index("s")
    w = c * NS + s
    pltpu.sync_copy(srcs.at[w], src_v)
    pltpu.sync_copy(dsts.at[w], dst_v)
    pltpu.sync_copy(zeros.at[pl.ds(s * SLAB, SLAB)],
                    acc.at[pl.ds(s * SLAB, SLAB)])
    plsc.subcore_barrier()

    def step(j, carry):
        del carry

        def gather_into(buf):
            pltpu.sync_copy(feat.at[src_v.at[j]], buf)

        def scatter_from(buf):
            pltpu.sync_copy(buf, acc.at[dst_v.at[j]], add=True)

        @pl.when(j % 2 == 0)
        def _():
            gather_into(rows_a)
            scatter_from(rows_a)

        @pl.when(j % 2 == 1)
        def _():
            gather_into(rows_b)
            scatter_from(rows_b)

        return 0

    lax.fori_loop(0, CHUNKS, step, 0)
    plsc.subcore_barrier()
    pltpu.sync_copy(acc.at[pl.ds(s * SLAB, SLAB)],
                    out.at[c, pl.ds(s * SLAB, SLAB)])


@jax.jit
def _segment_sum_sc(feat, srcs, dsts, zeros):
    mesh = plsc.VectorSubcoreMesh(core_axis_name="c", subcore_axis_name="s")
    return pl.kernel(
        _seg_body,
        mesh=mesh,
        out_type=jax.ShapeDtypeStruct((NC, N_ACC, F), jnp.float32),
        scratch_types=[
            pltpu.VMEM((CHUNKS, CHUNK), jnp.int32),
            pltpu.VMEM((CHUNKS, CHUNK), jnp.int32),
            pltpu.VMEM((CHUNK, F), jnp.float32),
            pltpu.VMEM((CHUNK, F), jnp.float32),
            pltpu.VMEM_SHARED((N_ACC, F), jnp.float32),
        ],
    )(feat, srcs, dsts, zeros)


def _mm_body(x_ref, w_ref, o_ref):
    o_ref[...] = jnp.dot(x_ref[...], w_ref[...],
                         preferred_element_type=jnp.float32)


@jax.jit
def _x_w1(x, W1):
    return pl.pallas_call(
        _mm_body,
        grid=(10,),
        in_specs=[
            pl.BlockSpec((N // 10, 128), lambda i: (i, 0)),
            pl.BlockSpec((128, F), lambda i: (0, 0)),
        ],
        out_specs=pl.BlockSpec((N // 10, F), lambda i: (i, 0)),
        out_shape=jax.ShapeDtypeStruct((N, F), jnp.float32),
    )(x, W1)


def _comb_body(p_ref, b_ref, o_ref):
    o_ref[...] = jnp.maximum(p_ref[0] + p_ref[1] + b_ref[...], 0.0)


@jax.jit
def _combine_relu(p, b):
    return pl.pallas_call(
        _comb_body,
        grid=(10,),
        in_specs=[
            pl.BlockSpec((2, N // 10, F), lambda i: (0, i, 0)),
            pl.BlockSpec((1, F), lambda i: (0, 0)),
        ],
        out_specs=pl.BlockSpec((N // 10, F), lambda i: (i, 0)),
        out_shape=jax.ShapeDtypeStruct((N, F), jnp.float32),
    )(p[:, :N, :], b.reshape(1, F))


def _head_body(p_ref, b1_ref, w2_ref, b2_ref, wd_ref, bd_ref, wo_ref, bo_ref,
               o_ref):
    agg = jnp.maximum(p_ref[0] + p_ref[1] + b1_ref[...], 0.0)
    h2 = jnp.maximum(
        jnp.dot(agg, w2_ref[...], preferred_element_type=jnp.float32)
        + b2_ref[...], 0.0)
    h3 = jnp.maximum(
        jnp.dot(h2, wd_ref[...], preferred_element_type=jnp.float32)
        + bd_ref[...], 0.0)
    z = jnp.dot(h3, wo_ref[...], preferred_element_type=jnp.float32) \
        + bo_ref[...]
    o_ref[...] = 1.0 / (1.0 + jnp.exp(-z))


@jax.jit
def _head(p, b1, W2, b2, Wd, bd, Wo, bo):
    blk = N // 10
    return pl.pallas_call(
        _head_body,
        grid=(10,),
        in_specs=[
            pl.BlockSpec((2, blk, F), lambda i: (0, i, 0)),
            pl.BlockSpec((1, F), lambda i: (0, 0)),
            pl.BlockSpec((F, 64), lambda i: (0, 0)),
            pl.BlockSpec((1, 64), lambda i: (0, 0)),
            pl.BlockSpec((64, 128), lambda i: (0, 0)),
            pl.BlockSpec((1, 128), lambda i: (0, 0)),
            pl.BlockSpec((128, 1), lambda i: (0, 0)),
            pl.BlockSpec((1, 1), lambda i: (0, 0)),
        ],
        out_specs=pl.BlockSpec((blk, 1), lambda i: (i, 0)),
        out_shape=jax.ShapeDtypeStruct((N, 1), jnp.float32),
    )(p[:, :N, :], b1.reshape(1, F), W2, b2.reshape(1, 64), Wd,
      bd.reshape(1, 128), Wo, bo.reshape(1, 1))


def kernel(x, edge_index, W1, b1, W2, b2, Wd, bd, Wo, bo):
    src = edge_index[0].astype(jnp.int32)
    dst = edge_index[1].astype(jnp.int32)
    pad = E_PAD - E
    src = jnp.concatenate([src, jnp.zeros((pad,), jnp.int32)])
    dst = jnp.concatenate([dst, jnp.full((pad,), N, jnp.int32)])
    srcs = src.reshape(NW, CHUNKS, CHUNK)
    dsts = dst.reshape(NW, CHUNKS, CHUNK)
    zeros = jnp.zeros((N_ACC, F), jnp.float32)

    t1 = _x_w1(x, W1)
    p1 = _segment_sum_sc(t1, srcs, dsts, zeros)
    h1 = _combine_relu(p1, b1)
    p2 = _segment_sum_sc(h1, srcs, dsts, zeros)
    out = _head(p2, b1 * 0.0 + 0.0, W2, b2, Wd, bd, Wo, bo)
    return out


# speedup vs baseline: 7.2392x; 7.2392x over previous
import functools

import jax
import jax.numpy as jnp
from jax import lax
from jax.experimental import pallas as pl
from jax.experimental.pallas import tpu as pltpu
from jax.experimental.pallas import tpu_sc as plsc

N = 10000
E = 320000
F = 32

NC = 2
NS = 16
NW = NC * NS
CHUNK = 128
CHUNKS = 80
E_PAD = NW * CHUNKS * CHUNK
N_ACC = 10112
SLAB = N_ACC // NS


def _seg_body(feat, srcs, dsts, zeros, out, src_v, dst_v, rows_a, rows_b, acc):
    c = lax.axis_index("c")
    s = lax.axis_index("s")
    w = c * NS + s
    pltpu.sync_copy(srcs.at[w], src_v)
    pltpu.sync_copy(dsts.at[w], dst_v)
    pltpu.sync_copy(zeros.at[pl.ds(s * SLAB, SLAB)],
                    acc.at[pl.ds(s * SLAB, SLAB)])
    plsc.subcore_barrier()

    def step(j, carry):
        del carry

        def gather_into(buf):
            pltpu.sync_copy(feat.at[src_v.at[j]], buf)

        def scatter_from(buf):
            pltpu.sync_copy(buf, acc.at[dst_v.at[j]], add=True)

        @pl.when(j % 2 == 0)
        def _():
            gather_into(rows_a)
            scatter_from(rows_a)

        @pl.when(j % 2 == 1)
        def _():
            gather_into(rows_b)
            scatter_from(rows_b)

        return 0

    lax.fori_loop(0, CHUNKS, step, 0)
    plsc.subcore_barrier()
    pltpu.sync_copy(acc.at[pl.ds(s * SLAB, SLAB)],
                    out.at[c, pl.ds(s * SLAB, SLAB)])


@jax.jit
def _segment_sum_sc(feat, srcs, dsts, zeros):
    mesh = plsc.VectorSubcoreMesh(core_axis_name="c", subcore_axis_name="s")
    return pl.kernel(
        _seg_body,
        mesh=mesh,
        compiler_params=pltpu.CompilerParams(use_tc_tiling_on_sc=False),
        out_type=jax.ShapeDtypeStruct((NC, N_ACC, F), jnp.float32),
        scratch_types=[
            pltpu.VMEM((CHUNKS, CHUNK), jnp.int32),
            pltpu.VMEM((CHUNKS, CHUNK), jnp.int32),
            pltpu.VMEM((CHUNK, F), jnp.float32),
            pltpu.VMEM((CHUNK, F), jnp.float32),
            pltpu.VMEM_SHARED((N_ACC, F), jnp.float32),
        ],
    )(feat, srcs, dsts, zeros)


def _mm_body(x_ref, w_ref, o_ref):
    o_ref[...] = jnp.dot(x_ref[...], w_ref[...],
                         preferred_element_type=jnp.float32)


@jax.jit
def _x_w1(x, W1):
    return pl.pallas_call(
        _mm_body,
        grid=(10,),
        in_specs=[
            pl.BlockSpec((N // 10, 128), lambda i: (i, 0)),
            pl.BlockSpec((128, F), lambda i: (0, 0)),
        ],
        out_specs=pl.BlockSpec((N // 10, F), lambda i: (i, 0)),
        out_shape=jax.ShapeDtypeStruct((N, F), jnp.float32),
    )(x, W1)


def _comb_body(p_ref, b_ref, o_ref):
    o_ref[...] = jnp.maximum(p_ref[0] + p_ref[1] + b_ref[...], 0.0)


@jax.jit
def _combine_relu(p, b):
    return pl.pallas_call(
        _comb_body,
        grid=(10,),
        in_specs=[
            pl.BlockSpec((2, N // 10, F), lambda i: (0, i, 0)),
            pl.BlockSpec((1, F), lambda i: (0, 0)),
        ],
        out_specs=pl.BlockSpec((N // 10, F), lambda i: (i, 0)),
        out_shape=jax.ShapeDtypeStruct((N, F), jnp.float32),
    )(p[:, :N, :], b.reshape(1, F))


def _head_body(p_ref, w2_ref, b2_ref, wd_ref, bd_ref, wo_ref, bo_ref,
               o_ref):
    agg = p_ref[0] + p_ref[1]
    h2 = jnp.maximum(
        jnp.dot(agg, w2_ref[...], preferred_element_type=jnp.float32)
        + b2_ref[...], 0.0)
    h3 = jnp.maximum(
        jnp.dot(h2, wd_ref[...], preferred_element_type=jnp.float32)
        + bd_ref[...], 0.0)
    z = jnp.dot(h3, wo_ref[...], preferred_element_type=jnp.float32) \
        + bo_ref[...]
    o_ref[...] = 1.0 / (1.0 + jnp.exp(-z))


@jax.jit
def _head(p, W2, b2, Wd, bd, Wo, bo):
    blk = N // 10
    return pl.pallas_call(
        _head_body,
        grid=(10,),
        in_specs=[
            pl.BlockSpec((2, blk, F), lambda i: (0, i, 0)),
            pl.BlockSpec((F, 64), lambda i: (0, 0)),
            pl.BlockSpec((1, 64), lambda i: (0, 0)),
            pl.BlockSpec((64, 128), lambda i: (0, 0)),
            pl.BlockSpec((1, 128), lambda i: (0, 0)),
            pl.BlockSpec((128, 1), lambda i: (0, 0)),
            pl.BlockSpec((1, 1), lambda i: (0, 0)),
        ],
        out_specs=pl.BlockSpec((blk, 1), lambda i: (i, 0)),
        out_shape=jax.ShapeDtypeStruct((N, 1), jnp.float32),
    )(p[:, :N, :], W2, b2.reshape(1, 64), Wd,
      bd.reshape(1, 128), Wo, bo.reshape(1, 1))


def kernel(x, edge_index, W1, b1, W2, b2, Wd, bd, Wo, bo):
    src = edge_index[0].astype(jnp.int32)
    dst = edge_index[1].astype(jnp.int32)
    pad = E_PAD - E
    src = jnp.concatenate([src, jnp.zeros((pad,), jnp.int32)])
    dst = jnp.concatenate([dst, jnp.full((pad,), N, jnp.int32)])
    srcs = src.reshape(NW, CHUNKS, CHUNK)
    dsts = dst.reshape(NW, CHUNKS, CHUNK)
    zeros = jnp.zeros((N_ACC, F), jnp.float32)

    t1 = _x_w1(x, W1)
    p1 = _segment_sum_sc(t1, srcs, dsts, zeros)
    h1 = _combine_relu(p1, b1)
    p2 = _segment_sum_sc(h1, srcs, dsts, zeros)
    out = _head(p2, W2, b2, Wd, bd, Wo, bo)
    return out


# speedup vs baseline: 8.8302x; 1.2198x over previous
import functools

import jax
import jax.numpy as jnp
from jax import lax
from jax.experimental import pallas as pl
from jax.experimental.pallas import tpu as pltpu
from jax.experimental.pallas import tpu_sc as plsc

N = 10000
E = 320000
F = 32

NC = 2
NS = 16
NW = NC * NS
CHUNK = 128
CHUNKS = 80
E_PAD = NW * CHUNKS * CHUNK
N_ACC = 10112
SLAB = N_ACC // NS


NBUF = 4
SUPER = CHUNKS // NBUF


def _seg_body(feat, srcs, dsts, zeros, out, src_v, dst_v, rows, acc,
              gsems, ssems):
    c = lax.axis_index("c")
    s = lax.axis_index("s")
    w = c * NS + s
    pltpu.sync_copy(srcs.at[w], src_v)
    pltpu.sync_copy(dsts.at[w], dst_v)
    pltpu.sync_copy(zeros.at[pl.ds(s * SLAB, SLAB)],
                    acc.at[pl.ds(s * SLAB, SLAB)])
    plsc.subcore_barrier()

    def outer(J, carry):
        del carry
        for b in range(NBUF):
            j = J * NBUF + b

            @pl.when(J > 0)
            def _(b=b):
                pltpu.make_async_copy(feat.at[pl.ds(0, CHUNK)], rows.at[b],
                                      ssems.at[b]).wait()
            pltpu.make_async_copy(feat.at[src_v.at[j]], rows.at[b],
                                  gsems.at[b]).start()
        for b in range(NBUF):
            j = J * NBUF + b
            pltpu.make_async_copy(feat.at[pl.ds(0, CHUNK)], rows.at[b],
                                  gsems.at[b]).wait()
            pltpu.async_copy(rows.at[b], acc.at[dst_v.at[j]],
                             ssems.at[b], add=True)
        return 0

    lax.fori_loop(0, SUPER, outer, 0)
    for b in range(NBUF):
        pltpu.make_async_copy(feat.at[pl.ds(0, CHUNK)], rows.at[b],
                              ssems.at[b]).wait()
    plsc.subcore_barrier()
    pltpu.sync_copy(acc.at[pl.ds(s * SLAB, SLAB)],
                    out.at[c, pl.ds(s * SLAB, SLAB)])


@jax.jit
def _segment_sum_sc(feat, srcs, dsts, zeros):
    mesh = plsc.VectorSubcoreMesh(core_axis_name="c", subcore_axis_name="s")
    return pl.kernel(
        _seg_body,
        mesh=mesh,
        compiler_params=pltpu.CompilerParams(use_tc_tiling_on_sc=False),
        out_type=jax.ShapeDtypeStruct((NC, N_ACC, F), jnp.float32),
        scratch_types=[
            pltpu.VMEM((CHUNKS, CHUNK), jnp.int32),
            pltpu.VMEM((CHUNKS, CHUNK), jnp.int32),
            pltpu.VMEM((NBUF, CHUNK, F), jnp.float32),
            pltpu.VMEM_SHARED((N_ACC, F), jnp.float32),
            pltpu.SemaphoreType.DMA((NBUF,)),
            pltpu.SemaphoreType.DMA((NBUF,)),
        ],
    )(feat, srcs, dsts, zeros)


def _mm_body(x_ref, w_ref, o_ref):
    o_ref[...] = jnp.dot(x_ref[...], w_ref[...],
                         preferred_element_type=jnp.float32)


@jax.jit
def _x_w1(x, W1):
    return pl.pallas_call(
        _mm_body,
        grid=(10,),
        in_specs=[
            pl.BlockSpec((N // 10, 128), lambda i: (i, 0)),
            pl.BlockSpec((128, F), lambda i: (0, 0)),
        ],
        out_specs=pl.BlockSpec((N // 10, F), lambda i: (i, 0)),
        out_shape=jax.ShapeDtypeStruct((N, F), jnp.float32),
    )(x, W1)


def _comb_body(p_ref, b_ref, o_ref):
    o_ref[...] = jnp.maximum(p_ref[0] + p_ref[1] + b_ref[...], 0.0)


@jax.jit
def _combine_relu(p, b):
    return pl.pallas_call(
        _comb_body,
        grid=(10,),
        in_specs=[
            pl.BlockSpec((2, N // 10, F), lambda i: (0, i, 0)),
            pl.BlockSpec((1, F), lambda i: (0, 0)),
        ],
        out_specs=pl.BlockSpec((N // 10, F), lambda i: (i, 0)),
        out_shape=jax.ShapeDtypeStruct((N, F), jnp.float32),
    )(p[:, :N, :], b.reshape(1, F))


def _head_body(p_ref, w2_ref, b2_ref, wd_ref, bd_ref, wo_ref, bo_ref,
               o_ref):
    agg = p_ref[0] + p_ref[1]
    h2 = jnp.maximum(
        jnp.dot(agg, w2_ref[...], preferred_element_type=jnp.float32)
        + b2_ref[...], 0.0)
    h3 = jnp.maximum(
        jnp.dot(h2, wd_ref[...], preferred_element_type=jnp.float32)
        + bd_ref[...], 0.0)
    z = jnp.dot(h3, wo_ref[...], preferred_element_type=jnp.float32) \
        + bo_ref[...]
    o_ref[...] = 1.0 / (1.0 + jnp.exp(-z))


@jax.jit
def _head(p, W2, b2, Wd, bd, Wo, bo):
    blk = N // 10
    return pl.pallas_call(
        _head_body,
        grid=(10,),
        in_specs=[
            pl.BlockSpec((2, blk, F), lambda i: (0, i, 0)),
            pl.BlockSpec((F, 64), lambda i: (0, 0)),
            pl.BlockSpec((1, 64), lambda i: (0, 0)),
            pl.BlockSpec((64, 128), lambda i: (0, 0)),
            pl.BlockSpec((1, 128), lambda i: (0, 0)),
            pl.BlockSpec((128, 1), lambda i: (0, 0)),
            pl.BlockSpec((1, 1), lambda i: (0, 0)),
        ],
        out_specs=pl.BlockSpec((blk, 1), lambda i: (i, 0)),
        out_shape=jax.ShapeDtypeStruct((N, 1), jnp.float32),
    )(p[:, :N, :], W2, b2.reshape(1, 64), Wd,
      bd.reshape(1, 128), Wo, bo.reshape(1, 1))


def kernel(x, edge_index, W1, b1, W2, b2, Wd, bd, Wo, bo):
    src = edge_index[0].astype(jnp.int32)
    dst = edge_index[1].astype(jnp.int32)
    pad = E_PAD - E
    src = jnp.concatenate([src, jnp.zeros((pad,), jnp.int32)])
    dst = jnp.concatenate([dst, jnp.full((pad,), N, jnp.int32)])
    srcs = src.reshape(NW, CHUNKS, CHUNK)
    dsts = dst.reshape(NW, CHUNKS, CHUNK)
    zeros = jnp.zeros((N_ACC, F), jnp.float32)

    t1 = _x_w1(x, W1)
    p1 = _segment_sum_sc(t1, srcs, dsts, zeros)
    h1 = _combine_relu(p1, b1)
    p2 = _segment_sum_sc(h1, srcs, dsts, zeros)
    out = _head(p2, W2, b2, Wd, bd, Wo, bo)
    return out


# speedup vs baseline: 15.5760x; 1.7639x over previous
import functools

import jax
import jax.numpy as jnp
from jax import lax
from jax.experimental import pallas as pl
from jax.experimental.pallas import tpu as pltpu
from jax.experimental.pallas import tpu_sc as plsc

N = 10000
E = 320000
F = 32

NC = 2
NS = 16
NW = NC * NS
CHUNK = 128
CHUNKS = 80
E_PAD = NW * CHUNKS * CHUNK
N_ACC = 10112
SLAB = N_ACC // NS


NBUF = 4
SUPER = CHUNKS // NBUF


def _seg_body(feat, srcs, dsts, zeros, out, src_v, dst_v, rows, acc, feat_sh,
              gsems, ssems):
    c = lax.axis_index("c")
    s = lax.axis_index("s")
    w = c * NS + s
    pltpu.sync_copy(srcs.at[w], src_v)
    pltpu.sync_copy(dsts.at[w], dst_v)
    pltpu.sync_copy(feat.at[pl.ds(s * SLAB, SLAB)],
                    feat_sh.at[pl.ds(s * SLAB, SLAB)])
    pltpu.sync_copy(zeros.at[pl.ds(s * SLAB, SLAB)],
                    acc.at[pl.ds(s * SLAB, SLAB)])
    plsc.subcore_barrier()

    def outer(J, carry):
        del carry
        for b in range(NBUF):
            j = J * NBUF + b

            @pl.when(J > 0)
            def _(b=b):
                pltpu.make_async_copy(feat.at[pl.ds(0, CHUNK)], rows.at[b],
                                      ssems.at[b]).wait()
            pltpu.make_async_copy(feat_sh.at[src_v.at[j]], rows.at[b],
                                  gsems.at[b]).start()
        for b in range(NBUF):
            j = J * NBUF + b
            pltpu.make_async_copy(feat.at[pl.ds(0, CHUNK)], rows.at[b],
                                  gsems.at[b]).wait()
            pltpu.async_copy(rows.at[b], acc.at[dst_v.at[j]],
                             ssems.at[b], add=True)
        return 0

    lax.fori_loop(0, SUPER, outer, 0)
    for b in range(NBUF):
        pltpu.make_async_copy(feat.at[pl.ds(0, CHUNK)], rows.at[b],
                              ssems.at[b]).wait()
    plsc.subcore_barrier()
    pltpu.sync_copy(acc.at[pl.ds(s * SLAB, SLAB)],
                    out.at[c, pl.ds(s * SLAB, SLAB)])


@jax.jit
def _segment_sum_sc(feat, srcs, dsts, zeros):
    mesh = plsc.VectorSubcoreMesh(core_axis_name="c", subcore_axis_name="s")
    return pl.kernel(
        _seg_body,
        mesh=mesh,
        compiler_params=pltpu.CompilerParams(use_tc_tiling_on_sc=False),
        out_type=jax.ShapeDtypeStruct((NC, N_ACC, F), jnp.float32),
        scratch_types=[
            pltpu.VMEM((CHUNKS, CHUNK), jnp.int32),
            pltpu.VMEM((CHUNKS, CHUNK), jnp.int32),
            pltpu.VMEM((NBUF, CHUNK, F), jnp.float32),
            pltpu.VMEM_SHARED((N_ACC, F), jnp.float32),
            pltpu.VMEM_SHARED((N_ACC, F), jnp.float32),
            pltpu.SemaphoreType.DMA((NBUF,)),
            pltpu.SemaphoreType.DMA((NBUF,)),
        ],
    )(feat, srcs, dsts, zeros)


def _mm_body(x_ref, w_ref, o_ref):
    o_ref[...] = jnp.dot(x_ref[...], w_ref[...],
                         preferred_element_type=jnp.float32)


@jax.jit
def _x_w1(x, W1):
    return pl.pallas_call(
        _mm_body,
        grid=(NS,),
        in_specs=[
            pl.BlockSpec((SLAB, 128), lambda i: (i, 0)),
            pl.BlockSpec((128, F), lambda i: (0, 0)),
        ],
        out_specs=pl.BlockSpec((SLAB, F), lambda i: (i, 0)),
        out_shape=jax.ShapeDtypeStruct((N_ACC, F), jnp.float32),
    )(x, W1)


def _comb_body(p_ref, b_ref, o_ref):
    o_ref[...] = jnp.maximum(p_ref[0] + p_ref[1] + b_ref[...], 0.0)


@jax.jit
def _combine_relu(p, b):
    return pl.pallas_call(
        _comb_body,
        grid=(NS,),
        in_specs=[
            pl.BlockSpec((2, SLAB, F), lambda i: (0, i, 0)),
            pl.BlockSpec((1, F), lambda i: (0, 0)),
        ],
        out_specs=pl.BlockSpec((SLAB, F), lambda i: (i, 0)),
        out_shape=jax.ShapeDtypeStruct((N_ACC, F), jnp.float32),
    )(p, b.reshape(1, F))


def _head_body(p_ref, w2_ref, b2_ref, wd_ref, bd_ref, wo_ref, bo_ref,
               o_ref):
    agg = p_ref[0] + p_ref[1]
    h2 = jnp.maximum(
        jnp.dot(agg, w2_ref[...], preferred_element_type=jnp.float32)
        + b2_ref[...], 0.0)
    h3 = jnp.maximum(
        jnp.dot(h2, wd_ref[...], preferred_element_type=jnp.float32)
        + bd_ref[...], 0.0)
    z = jnp.dot(h3, wo_ref[...], preferred_element_type=jnp.float32) \
        + bo_ref[...]
    o_ref[...] = 1.0 / (1.0 + jnp.exp(-z))


@jax.jit
def _head(p, W2, b2, Wd, bd, Wo, bo):
    blk = SLAB
    return pl.pallas_call(
        _head_body,
        grid=(NS,),
        in_specs=[
            pl.BlockSpec((2, blk, F), lambda i: (0, i, 0)),
            pl.BlockSpec((F, 64), lambda i: (0, 0)),
            pl.BlockSpec((1, 64), lambda i: (0, 0)),
            pl.BlockSpec((64, 128), lambda i: (0, 0)),
            pl.BlockSpec((1, 128), lambda i: (0, 0)),
            pl.BlockSpec((128, 1), lambda i: (0, 0)),
            pl.BlockSpec((1, 1), lambda i: (0, 0)),
        ],
        out_specs=pl.BlockSpec((blk, 1), lambda i: (i, 0)),
        out_shape=jax.ShapeDtypeStruct((N_ACC, 1), jnp.float32),
    )(p, W2, b2.reshape(1, 64), Wd,
      bd.reshape(1, 128), Wo, bo.reshape(1, 1))


def kernel(x, edge_index, W1, b1, W2, b2, Wd, bd, Wo, bo):
    src = edge_index[0].astype(jnp.int32)
    dst = edge_index[1].astype(jnp.int32)
    pad = E_PAD - E
    src = jnp.concatenate([src, jnp.zeros((pad,), jnp.int32)])
    dst = jnp.concatenate([dst, jnp.full((pad,), N, jnp.int32)])
    srcs = src.reshape(NW, CHUNKS, CHUNK)
    dsts = dst.reshape(NW, CHUNKS, CHUNK)
    zeros = jnp.zeros((N_ACC, F), jnp.float32)
    x_pad = jnp.pad(x, ((0, N_ACC - N), (0, 0)))

    t1 = _x_w1(x_pad, W1)
    p1 = _segment_sum_sc(t1, srcs, dsts, zeros)
    h1 = _combine_relu(p1, b1)
    p2 = _segment_sum_sc(h1, srcs, dsts, zeros)
    out = _head(p2, W2, b2, Wd, bd, Wo, bo)
    return out[:N]


# speedup vs baseline: 16.7236x; 1.0737x over previous
import functools

import jax
import jax.numpy as jnp
from jax import lax
from jax.experimental import pallas as pl
from jax.experimental.pallas import tpu as pltpu
from jax.experimental.pallas import tpu_sc as plsc

N = 10000
E = 320000
F = 32

NC = 2
NS = 16
NW = NC * NS
CHUNK = 128
CHUNKS = 80
E_PAD = NW * CHUNKS * CHUNK
N_ACC = 10112
SLAB = N_ACC // NS


NBUF = 4
SUPER = CHUNKS // NBUF


def _zero_fill(buf):
    zv = jnp.zeros((16,), jnp.float32)

    def zstep(k, carry):
        del carry
        buf[k, pl.ds(0, 16)] = zv
        buf[k, pl.ds(16, 16)] = zv
        return 0

    lax.fori_loop(0, SLAB, zstep, 0)


def _edge_pipeline(hbm_dummy, src_v, dst_v, rows, acc, feat_sh, gsems, ssems):
    def outer(J, carry):
        del carry
        for b in range(NBUF):
            j = J * NBUF + b

            @pl.when(J > 0)
            def _(b=b):
                pltpu.make_async_copy(hbm_dummy, rows.at[b],
                                      ssems.at[b]).wait()
            pltpu.make_async_copy(feat_sh.at[src_v.at[j]], rows.at[b],
                                  gsems.at[b]).start()
        for b in range(NBUF):
            j = J * NBUF + b
            pltpu.make_async_copy(hbm_dummy, rows.at[b], gsems.at[b]).wait()
            pltpu.async_copy(rows.at[b], acc.at[dst_v.at[j]],
                             ssems.at[b], add=True)
        return 0

    lax.fori_loop(0, SUPER, outer, 0)
    for b in range(NBUF):
        pltpu.make_async_copy(hbm_dummy, rows.at[b], ssems.at[b]).wait()


def _seg1_body(feat, srcs, dsts, out, src_v, dst_v, rows, zbuf, acc, feat_sh,
               gsems, ssems):
    c = lax.axis_index("c")
    s = lax.axis_index("s")
    w = c * NS + s
    pltpu.sync_copy(srcs.at[w], src_v)
    pltpu.sync_copy(dsts.at[w], dst_v)
    pltpu.sync_copy(feat.at[pl.ds(s * SLAB, SLAB)],
                    feat_sh.at[pl.ds(s * SLAB, SLAB)])
    _zero_fill(zbuf)
    pltpu.sync_copy(zbuf, acc.at[pl.ds(s * SLAB, SLAB)])
    plsc.subcore_barrier()
    _edge_pipeline(feat.at[pl.ds(0, CHUNK)], src_v, dst_v, rows, acc,
                   feat_sh, gsems, ssems)
    plsc.subcore_barrier()
    pltpu.sync_copy(acc.at[pl.ds(s * SLAB, SLAB)],
                    out.at[c, pl.ds(s * SLAB, SLAB)])


def _seg2_body(parts, srcs, dsts, bias, out, src_v, dst_v, rows, buf0, buf1,
               bvm, acc, feat_sh, gsems, ssems):
    c = lax.axis_index("c")
    s = lax.axis_index("s")
    w = c * NS + s
    pltpu.sync_copy(srcs.at[w], src_v)
    pltpu.sync_copy(dsts.at[w], dst_v)
    pltpu.sync_copy(parts.at[0, pl.ds(s * SLAB, SLAB)], buf0)
    pltpu.sync_copy(parts.at[1, pl.ds(s * SLAB, SLAB)], buf1)
    pltpu.sync_copy(bias, bvm)
    b_lo = bvm[pl.ds(0, 16)]
    b_hi = bvm[pl.ds(16, 16)]

    def cstep(k, carry):
        del carry
        v0 = buf0[k, pl.ds(0, 16)] + buf1[k, pl.ds(0, 16)] + b_lo
        buf0[k, pl.ds(0, 16)] = jnp.maximum(v0, 0.0)
        v1 = buf0[k, pl.ds(16, 16)] + buf1[k, pl.ds(16, 16)] + b_hi
        buf0[k, pl.ds(16, 16)] = jnp.maximum(v1, 0.0)
        return 0

    lax.fori_loop(0, SLAB, cstep, 0)
    pltpu.sync_copy(buf0, feat_sh.at[pl.ds(s * SLAB, SLAB)])
    _zero_fill(buf1)
    pltpu.sync_copy(buf1, acc.at[pl.ds(s * SLAB, SLAB)])
    plsc.subcore_barrier()
    _edge_pipeline(parts.at[0, pl.ds(0, CHUNK)], src_v, dst_v, rows, acc,
                   feat_sh, gsems, ssems)
    plsc.subcore_barrier()
    pltpu.sync_copy(acc.at[pl.ds(s * SLAB, SLAB)],
                    out.at[c, pl.ds(s * SLAB, SLAB)])


_SC_PARAMS = pltpu.CompilerParams(use_tc_tiling_on_sc=False)
_SC_MESH = dict(core_axis_name="c", subcore_axis_name="s")


@jax.jit
def _segment_sum_sc1(feat, srcs, dsts):
    return pl.kernel(
        _seg1_body,
        mesh=plsc.VectorSubcoreMesh(**_SC_MESH),
        compiler_params=_SC_PARAMS,
        out_type=jax.ShapeDtypeStruct((NC, N_ACC, F), jnp.float32),
        scratch_types=[
            pltpu.VMEM((CHUNKS, CHUNK), jnp.int32),
            pltpu.VMEM((CHUNKS, CHUNK), jnp.int32),
            pltpu.VMEM((NBUF, CHUNK, F), jnp.float32),
            pltpu.VMEM((SLAB, F), jnp.float32),
            pltpu.VMEM_SHARED((N_ACC, F), jnp.float32),
            pltpu.VMEM_SHARED((N_ACC, F), jnp.float32),
            pltpu.SemaphoreType.DMA((NBUF,)),
            pltpu.SemaphoreType.DMA((NBUF,)),
        ],
    )(feat, srcs, dsts)


@jax.jit
def _segment_sum_sc2(parts, srcs, dsts, bias):
    return pl.kernel(
        _seg2_body,
        mesh=plsc.VectorSubcoreMesh(**_SC_MESH),
        compiler_params=_SC_PARAMS,
        out_type=jax.ShapeDtypeStruct((NC, N_ACC, F), jnp.float32),
        scratch_types=[
            pltpu.VMEM((CHUNKS, CHUNK), jnp.int32),
            pltpu.VMEM((CHUNKS, CHUNK), jnp.int32),
            pltpu.VMEM((NBUF, CHUNK, F), jnp.float32),
            pltpu.VMEM((SLAB, F), jnp.float32),
            pltpu.VMEM((SLAB, F), jnp.float32),
            pltpu.VMEM((F,), jnp.float32),
            pltpu.VMEM_SHARED((N_ACC, F), jnp.float32),
            pltpu.VMEM_SHARED((N_ACC, F), jnp.float32),
            pltpu.SemaphoreType.DMA((NBUF,)),
            pltpu.SemaphoreType.DMA((NBUF,)),
        ],
    )(parts, srcs, dsts, bias)


def _mm_body(x_ref, w_ref, o_ref):
    o_ref[...] = jnp.dot(x_ref[...], w_ref[...],
                         preferred_element_type=jnp.float32)


@jax.jit
def _x_w1(x, W1):
    return pl.pallas_call(
        _mm_body,
        grid=(NS,),
        in_specs=[
            pl.BlockSpec((SLAB, 128), lambda i: (i, 0)),
            pl.BlockSpec((128, F), lambda i: (0, 0)),
        ],
        out_specs=pl.BlockSpec((SLAB, F), lambda i: (i, 0)),
        out_shape=jax.ShapeDtypeStruct((N_ACC, F), jnp.float32),
    )(x, W1)


def _head_body(p_ref, w2_ref, b2_ref, wd_ref, bd_ref, wo_ref, bo_ref,
               o_ref):
    agg = p_ref[0] + p_ref[1]
    h2 = jnp.maximum(
        jnp.dot(agg, w2_ref[...], preferred_element_type=jnp.float32)
        + b2_ref[...], 0.0)
    h3 = jnp.maximum(
        jnp.dot(h2, wd_ref[...], preferred_element_type=jnp.float32)
        + bd_ref[...], 0.0)
    z = jnp.dot(h3, wo_ref[...], preferred_element_type=jnp.float32) \
        + bo_ref[...]
    o_ref[...] = 1.0 / (1.0 + jnp.exp(-z))


@jax.jit
def _head(p, W2, b2, Wd, bd, Wo, bo):
    blk = SLAB
    return pl.pallas_call(
        _head_body,
        grid=(NS,),
        in_specs=[
            pl.BlockSpec((2, blk, F), lambda i: (0, i, 0)),
            pl.BlockSpec((F, 64), lambda i: (0, 0)),
            pl.BlockSpec((1, 64), lambda i: (0, 0)),
            pl.BlockSpec((64, 128), lambda i: (0, 0)),
            pl.BlockSpec((1, 128), lambda i: (0, 0)),
            pl.BlockSpec((128, 1), lambda i: (0, 0)),
            pl.BlockSpec((1, 1), lambda i: (0, 0)),
        ],
        out_specs=pl.BlockSpec((blk, 1), lambda i: (i, 0)),
        out_shape=jax.ShapeDtypeStruct((N_ACC, 1), jnp.float32),
    )(p, W2, b2.reshape(1, 64), Wd,
      bd.reshape(1, 128), Wo, bo.reshape(1, 1))


def kernel(x, edge_index, W1, b1, W2, b2, Wd, bd, Wo, bo):
    src = edge_index[0].astype(jnp.int32)
    dst = edge_index[1].astype(jnp.int32)
    pad = E_PAD - E
    src = jnp.concatenate([src, jnp.zeros((pad,), jnp.int32)])
    dst = jnp.concatenate([dst, jnp.full((pad,), N, jnp.int32)])
    srcs = src.reshape(NW, CHUNKS, CHUNK)
    dsts = dst.reshape(NW, CHUNKS, CHUNK)
    x_pad = jnp.pad(x, ((0, N_ACC - N), (0, 0)))

    t1 = _x_w1(x_pad, W1)
    p1 = _segment_sum_sc1(t1, srcs, dsts)
    p2 = _segment_sum_sc2(p1, srcs, dsts, b1)
    out = _head(p2, W2, b2, Wd, bd, Wo, bo)
    return out[:N]


# speedup vs baseline: 17.4232x; 1.0418x over previous
import functools

import jax
import jax.numpy as jnp
from jax import lax
from jax.experimental import pallas as pl
from jax.experimental.pallas import tpu as pltpu
from jax.experimental.pallas import tpu_sc as plsc

N = 10000
E = 320000
F = 32

NC = 2
NS = 16
NW = NC * NS
CHUNK = 128
CHUNKS = 80
E_PAD = NW * CHUNKS * CHUNK
N_ACC = 10112
SLAB = N_ACC // NS


NBUF = 4
SUPER = CHUNKS // NBUF


def _zero_fill(buf):
    zv = jnp.zeros((16,), jnp.float32)

    def zstep(k, carry):
        del carry
        buf[k, pl.ds(0, 16)] = zv
        buf[k, pl.ds(16, 16)] = zv
        return 0

    lax.fori_loop(0, SLAB, zstep, 0)


def _edge_pipeline(hbm_dummy, src_v, dst_v, rows, acc, feat_sh, gsems, ssems):
    def outer(J, carry):
        del carry
        for b in range(NBUF):
            j = J * NBUF + b

            @pl.when(J > 0)
            def _(b=b):
                pltpu.make_async_copy(hbm_dummy, rows.at[b],
                                      ssems.at[b]).wait()
            pltpu.make_async_copy(feat_sh.at[src_v.at[j]], rows.at[b],
                                  gsems.at[b]).start()
        for b in range(NBUF):
            j = J * NBUF + b
            pltpu.make_async_copy(hbm_dummy, rows.at[b], gsems.at[b]).wait()
            pltpu.async_copy(rows.at[b], acc.at[dst_v.at[j]],
                             ssems.at[b], add=True)
        return 0

    lax.fori_loop(0, SUPER, outer, 0)
    for b in range(NBUF):
        pltpu.make_async_copy(hbm_dummy, rows.at[b], ssems.at[b]).wait()


def _seg1_body(feat, srcs, dsts, out, src_v, dst_v, rows, zbuf, acc, feat_sh,
               gsems, ssems):
    c = lax.axis_index("c")
    s = lax.axis_index("s")
    w = c * NS + s
    pltpu.sync_copy(srcs.at[w], src_v)
    pltpu.sync_copy(dsts.at[w], dst_v)
    pltpu.sync_copy(feat.at[pl.ds(s * SLAB, SLAB)],
                    feat_sh.at[pl.ds(s * SLAB, SLAB)])
    _zero_fill(zbuf)
    pltpu.sync_copy(zbuf, acc.at[pl.ds(s * SLAB, SLAB)])
    plsc.subcore_barrier()
    _edge_pipeline(feat.at[pl.ds(0, CHUNK)], src_v, dst_v, rows, acc,
                   feat_sh, gsems, ssems)
    plsc.subcore_barrier()
    pltpu.sync_copy(acc.at[pl.ds(s * SLAB, SLAB)],
                    out.at[c, pl.ds(s * SLAB, SLAB)])


def _seg2_body(parts, srcs, dsts, bias, out, src_v, dst_v, rows, buf0, buf1,
               bvm, acc, feat_sh, gsems, ssems):
    c = lax.axis_index("c")
    s = lax.axis_index("s")
    w = c * NS + s
    pltpu.sync_copy(srcs.at[w], src_v)
    pltpu.sync_copy(dsts.at[w], dst_v)
    pltpu.sync_copy(parts.at[0, pl.ds(s * SLAB, SLAB)], buf0)
    pltpu.sync_copy(parts.at[1, pl.ds(s * SLAB, SLAB)], buf1)
    pltpu.sync_copy(bias, bvm)
    b_lo = bvm[pl.ds(0, 16)]
    b_hi = bvm[pl.ds(16, 16)]

    def cstep(k, carry):
        del carry
        v0 = buf0[k, pl.ds(0, 16)] + buf1[k, pl.ds(0, 16)] + b_lo
        buf0[k, pl.ds(0, 16)] = jnp.maximum(v0, 0.0)
        v1 = buf0[k, pl.ds(16, 16)] + buf1[k, pl.ds(16, 16)] + b_hi
        buf0[k, pl.ds(16, 16)] = jnp.maximum(v1, 0.0)
        return 0

    lax.fori_loop(0, SLAB, cstep, 0)
    pltpu.sync_copy(buf0, feat_sh.at[pl.ds(s * SLAB, SLAB)])
    _zero_fill(buf1)
    pltpu.sync_copy(buf1, acc.at[pl.ds(s * SLAB, SLAB)])
    plsc.subcore_barrier()
    _edge_pipeline(parts.at[0, pl.ds(0, CHUNK)], src_v, dst_v, rows, acc,
                   feat_sh, gsems, ssems)
    plsc.subcore_barrier()
    pltpu.sync_copy(acc.at[pl.ds(s * SLAB, SLAB)],
                    out.at[c, pl.ds(s * SLAB, SLAB)])


_SC_PARAMS = pltpu.CompilerParams(use_tc_tiling_on_sc=False)
_SC_MESH = dict(core_axis_name="c", subcore_axis_name="s")


@jax.jit
def _segment_sum_sc1(feat, srcs, dsts):
    return pl.kernel(
        _seg1_body,
        mesh=plsc.VectorSubcoreMesh(**_SC_MESH),
        compiler_params=_SC_PARAMS,
        out_type=jax.ShapeDtypeStruct((NC, N_ACC, F), jnp.float32),
        scratch_types=[
            pltpu.VMEM((CHUNKS, CHUNK), jnp.int32),
            pltpu.VMEM((CHUNKS, CHUNK), jnp.int32),
            pltpu.VMEM((NBUF, CHUNK, F), jnp.float32),
            pltpu.VMEM((SLAB, F), jnp.float32),
            pltpu.VMEM_SHARED((N_ACC, F), jnp.float32),
            pltpu.VMEM_SHARED((N_ACC, F), jnp.float32),
            pltpu.SemaphoreType.DMA((NBUF,)),
            pltpu.SemaphoreType.DMA((NBUF,)),
        ],
    )(feat, srcs, dsts)


@jax.jit
def _segment_sum_sc2(parts, srcs, dsts, bias):
    return pl.kernel(
        _seg2_body,
        mesh=plsc.VectorSubcoreMesh(**_SC_MESH),
        compiler_params=_SC_PARAMS,
        out_type=jax.ShapeDtypeStruct((NC, N_ACC, F), jnp.float32),
        scratch_types=[
            pltpu.VMEM((CHUNKS, CHUNK), jnp.int32),
            pltpu.VMEM((CHUNKS, CHUNK), jnp.int32),
            pltpu.VMEM((NBUF, CHUNK, F), jnp.float32),
            pltpu.VMEM((SLAB, F), jnp.float32),
            pltpu.VMEM((SLAB, F), jnp.float32),
            pltpu.VMEM((F,), jnp.float32),
            pltpu.VMEM_SHARED((N_ACC, F), jnp.float32),
            pltpu.VMEM_SHARED((N_ACC, F), jnp.float32),
            pltpu.SemaphoreType.DMA((NBUF,)),
            pltpu.SemaphoreType.DMA((NBUF,)),
        ],
    )(parts, srcs, dsts, bias)


def _mm_body(x_ref, w_ref, o_ref):
    o_ref[...] = jnp.dot(x_ref[...], w_ref[...],
                         preferred_element_type=jnp.float32)


@jax.jit
def _x_w1(x, W1):
    return pl.pallas_call(
        _mm_body,
        grid=(NS,),
        in_specs=[
            pl.BlockSpec((SLAB, 128), lambda i: (i, 0)),
            pl.BlockSpec((128, F), lambda i: (0, 0)),
        ],
        out_specs=pl.BlockSpec((SLAB, F), lambda i: (i, 0)),
        out_shape=jax.ShapeDtypeStruct((N_ACC, F), jnp.float32),
    )(x, W1)


def _head_body(p_ref, w2_ref, b2_ref, wd_ref, bd_ref, wo_ref, bo_ref,
               o_ref):
    agg = p_ref[0] + p_ref[1]
    h2 = jnp.maximum(
        jnp.dot(agg, w2_ref[...], preferred_element_type=jnp.float32)
        + b2_ref[...], 0.0)
    h3 = jnp.maximum(
        jnp.dot(h2, wd_ref[...], preferred_element_type=jnp.float32)
        + bd_ref[...], 0.0)
    z = jnp.dot(h3, wo_ref[...], preferred_element_type=jnp.float32) \
        + bo_ref[...]
    o_ref[...] = 1.0 / (1.0 + jnp.exp(-z))


@jax.jit
def _head(p, W2, b2, Wd, bd, Wo, bo):
    blk = N // 10
    return pl.pallas_call(
        _head_body,
        grid=(10,),
        in_specs=[
            pl.BlockSpec((2, blk, F), lambda i: (0, i, 0)),
            pl.BlockSpec((F, 64), lambda i: (0, 0)),
            pl.BlockSpec((1, 64), lambda i: (0, 0)),
            pl.BlockSpec((64, 128), lambda i: (0, 0)),
            pl.BlockSpec((1, 128), lambda i: (0, 0)),
            pl.BlockSpec((128, 1), lambda i: (0, 0)),
            pl.BlockSpec((1, 1), lambda i: (0, 0)),
        ],
        out_specs=pl.BlockSpec((blk, 1), lambda i: (i, 0)),
        out_shape=jax.ShapeDtypeStruct((N, 1), jnp.float32),
    )(p, W2, b2.reshape(1, 64), Wd,
      bd.reshape(1, 128), Wo, bo.reshape(1, 1))


def kernel(x, edge_index, W1, b1, W2, b2, Wd, bd, Wo, bo):
    src = edge_index[0].astype(jnp.int32)
    dst = edge_index[1].astype(jnp.int32)
    pad = E_PAD - E
    src = jnp.concatenate([src, jnp.zeros((pad,), jnp.int32)])
    dst = jnp.concatenate([dst, jnp.full((pad,), N, jnp.int32)])
    srcs = src.reshape(NW, CHUNKS, CHUNK)
    dsts = dst.reshape(NW, CHUNKS, CHUNK)

    t1 = _x_w1(x, W1)
    p1 = _segment_sum_sc1(t1, srcs, dsts)
    p2 = _segment_sum_sc2(p1, srcs, dsts, b1)
    return _head(p2, W2, b2, Wd, bd, Wo, bo)


# speedup vs baseline: 20.3964x; 1.1706x over previous
import jax
import jax.numpy as jnp
from jax import lax
from jax.experimental import pallas as pl
from jax.experimental.pallas import tpu as pltpu
from jax.experimental.pallas import tpu_sc as plsc

N = 10000
E = 320000
F = 32

NC = 2
NS = 16
NW = NC * NS
CHUNK = 128
ROWS_TOT = E // CHUNK
BASE_CHUNKS = ROWS_TOT // NW
EXTRA = ROWS_TOT - BASE_CHUNKS * NW
MAXCH = BASE_CHUNKS + 1

NBUF = 4
SUPER = BASE_CHUNKS // NBUF
RING = SUPER * NBUF

N_ACC = 10112
SLAB = N_ACC // NS
FSLAB_LAST = N - (NS - 1) * SLAB


def _zero_fill(buf):
    zv = jnp.zeros((16,), jnp.float32)

    def zstep(k, carry):
        del carry
        buf[k, pl.ds(0, 16)] = zv
        buf[k, pl.ds(16, 16)] = zv
        return 0

    lax.fori_loop(0, SLAB, zstep, 0)


def _stage_edges(ei, w, src_v, dst_v):
    base = BASE_CHUNKS * w + jnp.minimum(w, EXTRA)
    pltpu.sync_copy(ei.at[0, pl.ds(base, BASE_CHUNKS)],
                    src_v.at[pl.ds(0, BASE_CHUNKS)])
    pltpu.sync_copy(ei.at[1, pl.ds(base, BASE_CHUNKS)],
                    dst_v.at[pl.ds(0, BASE_CHUNKS)])

    @pl.when(w < EXTRA)
    def _():
        pltpu.sync_copy(ei.at[0, pl.ds(base + BASE_CHUNKS, 1)],
                        src_v.at[pl.ds(BASE_CHUNKS, 1)])
        pltpu.sync_copy(ei.at[1, pl.ds(base + BASE_CHUNKS, 1)],
                        dst_v.at[pl.ds(BASE_CHUNKS, 1)])
    return 78 + jnp.where(w < EXTRA, 1, 0)


def _edge_pipeline(hbm_dummy, nchunks, src_v, dst_v, rows, acc, feat_sh,
                   gsems, ssems):
    def outer(J, carry):
        del carry
        for b in range(NBUF):
            j = J * NBUF + b

            @pl.when(J > 0)
            def _(b=b):
                pltpu.make_async_copy(hbm_dummy, rows.at[b],
                                      ssems.at[b]).wait()
            pltpu.make_async_copy(feat_sh.at[src_v.at[j]], rows.at[b],
                                  gsems.at[b]).start()
        for b in range(NBUF):
            j = J * NBUF + b
            pltpu.make_async_copy(hbm_dummy, rows.at[b], gsems.at[b]).wait()
            pltpu.async_copy(rows.at[b], acc.at[dst_v.at[j]],
                             ssems.at[b], add=True)
        return 0

    lax.fori_loop(0, SUPER, outer, 0)
    for b in range(NBUF):
        pltpu.make_async_copy(hbm_dummy, rows.at[b], ssems.at[b]).wait()

    def tail(j, carry):
        del carry
        pltpu.sync_copy(feat_sh.at[src_v.at[j]], rows.at[0])
        pltpu.sync_copy(rows.at[0], acc.at[dst_v.at[j]], add=True)
        return 0

    lax.fori_loop(RING, nchunks, tail, 0)


def _stage_feat(feat, s, feat_sh):
    @pl.when(s < NS - 1)
    def _():
        pltpu.sync_copy(feat.at[pl.ds(s * SLAB, SLAB)],
                        feat_sh.at[pl.ds(s * SLAB, SLAB)])

    @pl.when(s == NS - 1)
    def _():
        pltpu.sync_copy(feat.at[pl.ds((NS - 1) * SLAB, FSLAB_LAST)],
                        feat_sh.at[pl.ds((NS - 1) * SLAB, FSLAB_LAST)])


def _seg1_body(feat, ei, out, src_v, dst_v, rows, zbuf, acc, feat_sh,
               gsems, ssems):
    c = lax.axis_index("c")
    s = lax.axis_index("s")
    w = c * NS + s
    nchunks = _stage_edges(ei, w, src_v, dst_v)
    _stage_feat(feat, s, feat_sh)
    _zero_fill(zbuf)
    pltpu.sync_copy(zbuf, acc.at[pl.ds(s * SLAB, SLAB)])
    plsc.subcore_barrier()
    _edge_pipeline(feat.at[pl.ds(0, CHUNK)], nchunks, src_v, dst_v, rows,
                   acc, feat_sh, gsems, ssems)
    plsc.subcore_barrier()
    pltpu.sync_copy(acc.at[pl.ds(s * SLAB, SLAB)],
                    out.at[c, pl.ds(s * SLAB, SLAB)])


def _seg2_body(parts, ei, bias, out, src_v, dst_v, rows, buf0, buf1, bvm,
               acc, feat_sh, gsems, ssems):
    c = lax.axis_index("c")
    s = lax.axis_index("s")
    w = c * NS + s
    nchunks = _stage_edges(ei, w, src_v, dst_v)
    pltpu.sync_copy(parts.at[0, pl.ds(s * SLAB, SLAB)], buf0)
    pltpu.sync_copy(parts.at[1, pl.ds(s * SLAB, SLAB)], buf1)
    pltpu.sync_copy(bias, bvm)
    b_lo = bvm[pl.ds(0, 16)]
    b_hi = bvm[pl.ds(16, 16)]

    def cstep(k, carry):
        del carry
        v0 = buf0[k, pl.ds(0, 16)] + buf1[k, pl.ds(0, 16)] + b_lo
        buf0[k, pl.ds(0, 16)] = jnp.maximum(v0, 0.0)
        v1 = buf0[k, pl.ds(16, 16)] + buf1[k, pl.ds(16, 16)] + b_hi
        buf0[k, pl.ds(16, 16)] = jnp.maximum(v1, 0.0)
        return 0

    lax.fori_loop(0, SLAB, cstep, 0)
    pltpu.sync_copy(buf0, feat_sh.at[pl.ds(s * SLAB, SLAB)])
    _zero_fill(buf1)
    pltpu.sync_copy(buf1, acc.at[pl.ds(s * SLAB, SLAB)])
    plsc.subcore_barrier()
    _edge_pipeline(parts.at[0, pl.ds(0, CHUNK)], nchunks, src_v, dst_v,
                   rows, acc, feat_sh, gsems, ssems)
    plsc.subcore_barrier()
    pltpu.sync_copy(acc.at[pl.ds(s * SLAB, SLAB)],
                    out.at[c, pl.ds(s * SLAB, SLAB)])


_SC_PARAMS = pltpu.CompilerParams(use_tc_tiling_on_sc=False)
_SC_MESH = dict(core_axis_name="c", subcore_axis_name="s")


@jax.jit
def _segment_sum_sc1(feat, ei):
    return pl.kernel(
        _seg1_body,
        mesh=plsc.VectorSubcoreMesh(**_SC_MESH),
        compiler_params=_SC_PARAMS,
        out_type=jax.ShapeDtypeStruct((NC, N_ACC, F), jnp.float32),
        scratch_types=[
            pltpu.VMEM((MAXCH, CHUNK), jnp.int32),
            pltpu.VMEM((MAXCH, CHUNK), jnp.int32),
            pltpu.VMEM((NBUF, CHUNK, F), jnp.float32),
            pltpu.VMEM((SLAB, F), jnp.float32),
            pltpu.VMEM_SHARED((N_ACC, F), jnp.float32),
            pltpu.VMEM_SHARED((N_ACC, F), jnp.float32),
            pltpu.SemaphoreType.DMA((NBUF,)),
            pltpu.SemaphoreType.DMA((NBUF,)),
        ],
    )(feat, ei)


@jax.jit
def _segment_sum_sc2(parts, ei, bias):
    return pl.kernel(
        _seg2_body,
        mesh=plsc.VectorSubcoreMesh(**_SC_MESH),
        compiler_params=_SC_PARAMS,
        out_type=jax.ShapeDtypeStruct((NC, N_ACC, F), jnp.float32),
        scratch_types=[
            pltpu.VMEM((MAXCH, CHUNK), jnp.int32),
            pltpu.VMEM((MAXCH, CHUNK), jnp.int32),
            pltpu.VMEM((NBUF, CHUNK, F), jnp.float32),
            pltpu.VMEM((SLAB, F), jnp.float32),
            pltpu.VMEM((SLAB, F), jnp.float32),
            pltpu.VMEM((F,), jnp.float32),
            pltpu.VMEM_SHARED((N_ACC, F), jnp.float32),
            pltpu.VMEM_SHARED((N_ACC, F), jnp.float32),
            pltpu.SemaphoreType.DMA((NBUF,)),
            pltpu.SemaphoreType.DMA((NBUF,)),
        ],
    )(parts, ei, bias)


def _mm_body(x_ref, w_ref, o_ref):
    o_ref[...] = jnp.dot(x_ref[...], w_ref[...],
                         preferred_element_type=jnp.float32)


@jax.jit
def _x_w1(x, W1):
    blk = N // 5
    return pl.pallas_call(
        _mm_body,
        grid=(5,),
        in_specs=[
            pl.BlockSpec((blk, 128), lambda i: (i, 0)),
            pl.BlockSpec((128, F), lambda i: (0, 0)),
        ],
        out_specs=pl.BlockSpec((blk, F), lambda i: (i, 0)),
        out_shape=jax.ShapeDtypeStruct((N, F), jnp.float32),
    )(x, W1)


def _head_body(p_ref, w2_ref, b2_ref, wd_ref, bd_ref, wo_ref, bo_ref,
               o_ref):
    agg = p_ref[0] + p_ref[1]
    h2 = jnp.maximum(
        jnp.dot(agg, w2_ref[...], preferred_element_type=jnp.float32)
        + b2_ref[...], 0.0)
    h3 = jnp.maximum(
        jnp.dot(h2, wd_ref[...], preferred_element_type=jnp.float32)
        + bd_ref[...], 0.0)
    z = jnp.dot(h3, wo_ref[...], preferred_element_type=jnp.float32) \
        + bo_ref[...]
    o_ref[...] = 1.0 / (1.0 + jnp.exp(-z))


@jax.jit
def _head(p, W2, b2, Wd, bd, Wo, bo):
    blk = N // 5
    return pl.pallas_call(
        _head_body,
        grid=(5,),
        in_specs=[
            pl.BlockSpec((2, blk, F), lambda i: (0, i, 0)),
            pl.BlockSpec((F, 64), lambda i: (0, 0)),
            pl.BlockSpec((1, 64), lambda i: (0, 0)),
            pl.BlockSpec((64, 128), lambda i: (0, 0)),
            pl.BlockSpec((1, 128), lambda i: (0, 0)),
            pl.BlockSpec((128, 1), lambda i: (0, 0)),
            pl.BlockSpec((1, 1), lambda i: (0, 0)),
        ],
        out_specs=pl.BlockSpec((blk, 1), lambda i: (i, 0)),
        out_shape=jax.ShapeDtypeStruct((N, 1), jnp.float32),
    )(p, W2, b2.reshape(1, 64), Wd,
      bd.reshape(1, 128), Wo, bo.reshape(1, 1))


def kernel(x, edge_index, W1, b1, W2, b2, Wd, bd, Wo, bo):
    ei = edge_index.astype(jnp.int32).reshape(2, ROWS_TOT, CHUNK)

    t1 = _x_w1(x, W1)
    p1 = _segment_sum_sc1(t1, ei)
    p2 = _segment_sum_sc2(p1, ei, b1)
    return _head(p2, W2, b2, Wd, bd, Wo, bo)


# speedup vs baseline: 21.1899x; 1.0389x over previous
import jax
import jax.numpy as jnp
from jax import lax
from jax.experimental import pallas as pl
from jax.experimental.pallas import tpu as pltpu
from jax.experimental.pallas import tpu_sc as plsc

N = 10000
E = 320000
F = 32

NC = 2
NS = 16
NW = NC * NS
CHUNK = 128
ROWS_TOT = E // CHUNK
BASE_CHUNKS = ROWS_TOT // NW
EXTRA = ROWS_TOT - BASE_CHUNKS * NW
MAXCH = BASE_CHUNKS + 1

NBUF = 4
SUPER = BASE_CHUNKS // NBUF
RING = SUPER * NBUF

N_ACC = 10112
SLAB = N_ACC // NS
FSLAB_LAST = N - (NS - 1) * SLAB


def _zero_fill(buf):
    zv = jnp.zeros((16,), jnp.float32)

    def zstep(k2, carry):
        del carry
        for u in range(2):
            buf[k2 * 2 + u, pl.ds(0, 16)] = zv
            buf[k2 * 2 + u, pl.ds(16, 16)] = zv
        return 0

    lax.fori_loop(0, SLAB // 2, zstep, 0)


def _stage_edges(ei, w, src_v, dst_v):
    base = BASE_CHUNKS * w + jnp.minimum(w, EXTRA)
    pltpu.sync_copy(ei.at[0, pl.ds(base, BASE_CHUNKS)],
                    src_v.at[pl.ds(0, BASE_CHUNKS)])
    pltpu.sync_copy(ei.at[1, pl.ds(base, BASE_CHUNKS)],
                    dst_v.at[pl.ds(0, BASE_CHUNKS)])

    @pl.when(w < EXTRA)
    def _():
        pltpu.sync_copy(ei.at[0, pl.ds(base + BASE_CHUNKS, 1)],
                        src_v.at[pl.ds(BASE_CHUNKS, 1)])
        pltpu.sync_copy(ei.at[1, pl.ds(base + BASE_CHUNKS, 1)],
                        dst_v.at[pl.ds(BASE_CHUNKS, 1)])
    return 78 + jnp.where(w < EXTRA, 1, 0)


def _edge_pipeline(hbm_dummy, nchunks, src_v, dst_v, rows, acc, feat_sh,
                   gsems, ssems):
    def outer(J, carry):
        del carry
        for b in range(NBUF):
            j = J * NBUF + b

            @pl.when(J > 0)
            def _(b=b):
                pltpu.make_async_copy(hbm_dummy, rows.at[b],
                                      ssems.at[b]).wait()
            pltpu.make_async_copy(feat_sh.at[src_v.at[j]], rows.at[b],
                                  gsems.at[b]).start()
        for b in range(NBUF):
            j = J * NBUF + b
            pltpu.make_async_copy(hbm_dummy, rows.at[b], gsems.at[b]).wait()
            pltpu.async_copy(rows.at[b], acc.at[dst_v.at[j]],
                             ssems.at[b], add=True)
        return 0

    lax.fori_loop(0, SUPER, outer, 0)
    for b in range(NBUF):
        pltpu.make_async_copy(hbm_dummy, rows.at[b], ssems.at[b]).wait()

    def tail(j, carry):
        del carry
        pltpu.sync_copy(feat_sh.at[src_v.at[j]], rows.at[0])
        pltpu.sync_copy(rows.at[0], acc.at[dst_v.at[j]], add=True)
        return 0

    lax.fori_loop(RING, nchunks, tail, 0)


def _stage_feat(feat, s, feat_sh):
    @pl.when(s < NS - 1)
    def _():
        pltpu.sync_copy(feat.at[pl.ds(s * SLAB, SLAB)],
                        feat_sh.at[pl.ds(s * SLAB, SLAB)])

    @pl.when(s == NS - 1)
    def _():
        pltpu.sync_copy(feat.at[pl.ds((NS - 1) * SLAB, FSLAB_LAST)],
                        feat_sh.at[pl.ds((NS - 1) * SLAB, FSLAB_LAST)])


def _seg1_body(feat, ei, out, src_v, dst_v, rows, zbuf, acc, feat_sh,
               gsems, ssems):
    c = lax.axis_index("c")
    s = lax.axis_index("s")
    w = c * NS + s
    nchunks = _stage_edges(ei, w, src_v, dst_v)
    _stage_feat(feat, s, feat_sh)
    _zero_fill(zbuf)
    pltpu.sync_copy(zbuf, acc.at[pl.ds(s * SLAB, SLAB)])
    plsc.subcore_barrier()
    _edge_pipeline(feat.at[pl.ds(0, CHUNK)], nchunks, src_v, dst_v, rows,
                   acc, feat_sh, gsems, ssems)
    plsc.subcore_barrier()
    pltpu.sync_copy(acc.at[pl.ds(s * SLAB, SLAB)],
                    out.at[c, pl.ds(s * SLAB, SLAB)])


def _seg2_body(parts, ei, bias, out, src_v, dst_v, rows, buf0, buf1, bvm,
               acc, feat_sh, gsems, ssems):
    c = lax.axis_index("c")
    s = lax.axis_index("s")
    w = c * NS + s
    nchunks = _stage_edges(ei, w, src_v, dst_v)
    pltpu.sync_copy(parts.at[0, pl.ds(s * SLAB, SLAB)], buf0)
    pltpu.sync_copy(parts.at[1, pl.ds(s * SLAB, SLAB)], buf1)
    pltpu.sync_copy(bias, bvm)
    b_lo = bvm[pl.ds(0, 16)]
    b_hi = bvm[pl.ds(16, 16)]

    def cstep(k2, carry):
        del carry
        for u in range(2):
            k = k2 * 2 + u
            v0 = buf0[k, pl.ds(0, 16)] + buf1[k, pl.ds(0, 16)] + b_lo
            buf0[k, pl.ds(0, 16)] = jnp.maximum(v0, 0.0)
            v1 = buf0[k, pl.ds(16, 16)] + buf1[k, pl.ds(16, 16)] + b_hi
            buf0[k, pl.ds(16, 16)] = jnp.maximum(v1, 0.0)
        return 0

    lax.fori_loop(0, SLAB // 2, cstep, 0)
    pltpu.sync_copy(buf0, feat_sh.at[pl.ds(s * SLAB, SLAB)])
    _zero_fill(buf1)
    pltpu.sync_copy(buf1, acc.at[pl.ds(s * SLAB, SLAB)])
    plsc.subcore_barrier()
    _edge_pipeline(parts.at[0, pl.ds(0, CHUNK)], nchunks, src_v, dst_v,
                   rows, acc, feat_sh, gsems, ssems)
    plsc.subcore_barrier()
    pltpu.sync_copy(acc.at[pl.ds(s * SLAB, SLAB)],
                    out.at[c, pl.ds(s * SLAB, SLAB)])


_SC_PARAMS = pltpu.CompilerParams(use_tc_tiling_on_sc=False)
_SC_MESH = dict(core_axis_name="c", subcore_axis_name="s")


@jax.jit
def _segment_sum_sc1(feat, ei):
    return pl.kernel(
        _seg1_body,
        mesh=plsc.VectorSubcoreMesh(**_SC_MESH),
        compiler_params=_SC_PARAMS,
        out_type=jax.ShapeDtypeStruct((NC, N_ACC, F), jnp.float32),
        scratch_types=[
            pltpu.VMEM((MAXCH, CHUNK), jnp.int32),
            pltpu.VMEM((MAXCH, CHUNK), jnp.int32),
            pltpu.VMEM((NBUF, CHUNK, F), jnp.float32),
            pltpu.VMEM((SLAB, F), jnp.float32),
            pltpu.VMEM_SHARED((N_ACC, F), jnp.float32),
            pltpu.VMEM_SHARED((N_ACC, F), jnp.float32),
            pltpu.SemaphoreType.DMA((NBUF,)),
            pltpu.SemaphoreType.DMA((NBUF,)),
        ],
    )(feat, ei)


@jax.jit
def _segment_sum_sc2(parts, ei, bias):
    return pl.kernel(
        _seg2_body,
        mesh=plsc.VectorSubcoreMesh(**_SC_MESH),
        compiler_params=_SC_PARAMS,
        out_type=jax.ShapeDtypeStruct((NC, N_ACC, F), jnp.float32),
        scratch_types=[
            pltpu.VMEM((MAXCH, CHUNK), jnp.int32),
            pltpu.VMEM((MAXCH, CHUNK), jnp.int32),
            pltpu.VMEM((NBUF, CHUNK, F), jnp.float32),
            pltpu.VMEM((SLAB, F), jnp.float32),
            pltpu.VMEM((SLAB, F), jnp.float32),
            pltpu.VMEM((F,), jnp.float32),
            pltpu.VMEM_SHARED((N_ACC, F), jnp.float32),
            pltpu.VMEM_SHARED((N_ACC, F), jnp.float32),
            pltpu.SemaphoreType.DMA((NBUF,)),
            pltpu.SemaphoreType.DMA((NBUF,)),
        ],
    )(parts, ei, bias)


def _mm_body(x_ref, w_ref, o_ref):
    o_ref[...] = jnp.dot(x_ref[...], w_ref[...],
                         preferred_element_type=jnp.float32)


@jax.jit
def _x_w1(x, W1):
    blk = N // 2
    return pl.pallas_call(
        _mm_body,
        grid=(2,),
        in_specs=[
            pl.BlockSpec((blk, 128), lambda i: (i, 0)),
            pl.BlockSpec((128, F), lambda i: (0, 0)),
        ],
        out_specs=pl.BlockSpec((blk, F), lambda i: (i, 0)),
        out_shape=jax.ShapeDtypeStruct((N, F), jnp.float32),
    )(x, W1)


def _head_body(p_ref, w2_ref, b2_ref, wd_ref, bd_ref, wo_ref, bo_ref,
               o_ref):
    agg = p_ref[0] + p_ref[1]
    h2 = jnp.maximum(
        jnp.dot(agg, w2_ref[...], preferred_element_type=jnp.float32)
        + b2_ref[...], 0.0)
    h3 = jnp.maximum(
        jnp.dot(h2, wd_ref[...], preferred_element_type=jnp.float32)
        + bd_ref[...], 0.0)
    z = jnp.dot(h3, wo_ref[...], preferred_element_type=jnp.float32) \
        + bo_ref[...]
    o_ref[...] = 1.0 / (1.0 + jnp.exp(-z))


@jax.jit
def _head(p, W2, b2, Wd, bd, Wo, bo):
    blk = N // 5
    return pl.pallas_call(
        _head_body,
        grid=(5,),
        in_specs=[
            pl.BlockSpec((2, blk, F), lambda i: (0, i, 0)),
            pl.BlockSpec((F, 64), lambda i: (0, 0)),
            pl.BlockSpec((1, 64), lambda i: (0, 0)),
            pl.BlockSpec((64, 128), lambda i: (0, 0)),
            pl.BlockSpec((1, 128), lambda i: (0, 0)),
            pl.BlockSpec((128, 1), lambda i: (0, 0)),
            pl.BlockSpec((1, 1), lambda i: (0, 0)),
        ],
        out_specs=pl.BlockSpec((blk, 1), lambda i: (i, 0)),
        out_shape=jax.ShapeDtypeStruct((N, 1), jnp.float32),
    )(p, W2, b2.reshape(1, 64), Wd,
      bd.reshape(1, 128), Wo, bo.reshape(1, 1))


def kernel(x, edge_index, W1, b1, W2, b2, Wd, bd, Wo, bo):
    ei = edge_index.astype(jnp.int32).reshape(2, ROWS_TOT, CHUNK)
    ei = lax.optimization_barrier(ei)

    t1 = _x_w1(x, W1)
    p1 = _segment_sum_sc1(t1, ei)
    p2 = _segment_sum_sc2(p1, ei, b1)
    return _head(p2, W2, b2, Wd, bd, Wo, bo)


# speedup vs baseline: 21.5630x; 1.0176x over previous
import jax
import jax.numpy as jnp
from jax import lax
from jax.experimental import pallas as pl
from jax.experimental.pallas import tpu as pltpu
from jax.experimental.pallas import tpu_sc as plsc

N = 10000
E = 320000
F = 32

NC = 2
NS = 16
NW = NC * NS
CHUNK = 128
ROWS_TOT = E // CHUNK
BASE_CHUNKS = ROWS_TOT // NW
EXTRA = ROWS_TOT - BASE_CHUNKS * NW
MAXCH = BASE_CHUNKS + 1

NBUF = 6
SUPER = BASE_CHUNKS // NBUF
RING = SUPER * NBUF

N_ACC = 10112
SLAB = N_ACC // NS
FSLAB_LAST = N - (NS - 1) * SLAB


def _zero_fill(buf):
    zv = jnp.zeros((16,), jnp.float32)

    def zstep(k2, carry):
        del carry
        for u in range(2):
            buf[k2 * 2 + u, pl.ds(0, 16)] = zv
            buf[k2 * 2 + u, pl.ds(16, 16)] = zv
        return 0

    lax.fori_loop(0, SLAB // 2, zstep, 0)


def _stage_edges(ei, w, src_v, dst_v):
    base = BASE_CHUNKS * w + jnp.minimum(w, EXTRA)
    pltpu.sync_copy(ei.at[0, pl.ds(base, BASE_CHUNKS)],
                    src_v.at[pl.ds(0, BASE_CHUNKS)])
    pltpu.sync_copy(ei.at[1, pl.ds(base, BASE_CHUNKS)],
                    dst_v.at[pl.ds(0, BASE_CHUNKS)])

    @pl.when(w < EXTRA)
    def _():
        pltpu.sync_copy(ei.at[0, pl.ds(base + BASE_CHUNKS, 1)],
                        src_v.at[pl.ds(BASE_CHUNKS, 1)])
        pltpu.sync_copy(ei.at[1, pl.ds(base + BASE_CHUNKS, 1)],
                        dst_v.at[pl.ds(BASE_CHUNKS, 1)])
    return 78 + jnp.where(w < EXTRA, 1, 0)


def _edge_pipeline(hbm_dummy, nchunks, src_v, dst_v, rows, acc, feat_sh,
                   gsems, ssems):
    def outer(J, carry):
        del carry
        for b in range(NBUF):
            j = J * NBUF + b

            @pl.when(J > 0)
            def _(b=b):
                pltpu.make_async_copy(hbm_dummy, rows.at[b],
                                      ssems.at[b]).wait()
            pltpu.make_async_copy(feat_sh.at[src_v.at[j]], rows.at[b],
                                  gsems.at[b]).start()
        for b in range(NBUF):
            j = J * NBUF + b
            pltpu.make_async_copy(hbm_dummy, rows.at[b], gsems.at[b]).wait()
            pltpu.async_copy(rows.at[b], acc.at[dst_v.at[j]],
                             ssems.at[b], add=True)
        return 0

    lax.fori_loop(0, SUPER, outer, 0)
    for b in range(NBUF):
        pltpu.make_async_copy(hbm_dummy, rows.at[b], ssems.at[b]).wait()

    def tail(j, carry):
        del carry
        pltpu.sync_copy(feat_sh.at[src_v.at[j]], rows.at[0])
        pltpu.sync_copy(rows.at[0], acc.at[dst_v.at[j]], add=True)
        return 0

    lax.fori_loop(RING, nchunks, tail, 0)


def _stage_feat(feat, s, feat_sh):
    @pl.when(s < NS - 1)
    def _():
        pltpu.sync_copy(feat.at[pl.ds(s * SLAB, SLAB)],
                        feat_sh.at[pl.ds(s * SLAB, SLAB)])

    @pl.when(s == NS - 1)
    def _():
        pltpu.sync_copy(feat.at[pl.ds((NS - 1) * SLAB, FSLAB_LAST)],
                        feat_sh.at[pl.ds((NS - 1) * SLAB, FSLAB_LAST)])


def _seg1_body(feat, ei, out, src_v, dst_v, rows, zbuf, acc, feat_sh,
               gsems, ssems):
    c = lax.axis_index("c")
    s = lax.axis_index("s")
    w = c * NS + s
    nchunks = _stage_edges(ei, w, src_v, dst_v)
    _stage_feat(feat, s, feat_sh)
    _zero_fill(zbuf)
    pltpu.sync_copy(zbuf, acc.at[pl.ds(s * SLAB, SLAB)])
    plsc.subcore_barrier()
    _edge_pipeline(feat.at[pl.ds(0, CHUNK)], nchunks, src_v, dst_v, rows,
                   acc, feat_sh, gsems, ssems)
    plsc.subcore_barrier()
    pltpu.sync_copy(acc.at[pl.ds(s * SLAB, SLAB)],
                    out.at[c, pl.ds(s * SLAB, SLAB)])


def _seg2_body(parts, ei, bias, out, src_v, dst_v, rows, buf0, buf1, bvm,
               acc, feat_sh, gsems, ssems):
    c = lax.axis_index("c")
    s = lax.axis_index("s")
    w = c * NS + s
    nchunks = _stage_edges(ei, w, src_v, dst_v)
    pltpu.sync_copy(parts.at[0, pl.ds(s * SLAB, SLAB)], buf0)
    pltpu.sync_copy(parts.at[1, pl.ds(s * SLAB, SLAB)], buf1)
    pltpu.sync_copy(bias, bvm)
    b_lo = bvm[pl.ds(0, 16)]
    b_hi = bvm[pl.ds(16, 16)]

    def cstep(k2, carry):
        del carry
        for u in range(2):
            k = k2 * 2 + u
            v0 = buf0[k, pl.ds(0, 16)] + buf1[k, pl.ds(0, 16)] + b_lo
            buf0[k, pl.ds(0, 16)] = jnp.maximum(v0, 0.0)
            v1 = buf0[k, pl.ds(16, 16)] + buf1[k, pl.ds(16, 16)] + b_hi
            buf0[k, pl.ds(16, 16)] = jnp.maximum(v1, 0.0)
        return 0

    lax.fori_loop(0, SLAB // 2, cstep, 0)
    pltpu.sync_copy(buf0, feat_sh.at[pl.ds(s * SLAB, SLAB)])
    _zero_fill(buf1)
    pltpu.sync_copy(buf1, acc.at[pl.ds(s * SLAB, SLAB)])
    plsc.subcore_barrier()
    _edge_pipeline(parts.at[0, pl.ds(0, CHUNK)], nchunks, src_v, dst_v,
                   rows, acc, feat_sh, gsems, ssems)
    plsc.subcore_barrier()
    pltpu.sync_copy(acc.at[pl.ds(s * SLAB, SLAB)],
                    out.at[c, pl.ds(s * SLAB, SLAB)])


_SC_PARAMS = pltpu.CompilerParams(use_tc_tiling_on_sc=False)
_SC_MESH = dict(core_axis_name="c", subcore_axis_name="s")


@jax.jit
def _segment_sum_sc1(feat, ei):
    return pl.kernel(
        _seg1_body,
        mesh=plsc.VectorSubcoreMesh(**_SC_MESH),
        compiler_params=_SC_PARAMS,
        out_type=jax.ShapeDtypeStruct((NC, N_ACC, F), jnp.float32),
        scratch_types=[
            pltpu.VMEM((MAXCH, CHUNK), jnp.int32),
            pltpu.VMEM((MAXCH, CHUNK), jnp.int32),
            pltpu.VMEM((NBUF, CHUNK, F), jnp.float32),
            pltpu.VMEM((SLAB, F), jnp.float32),
            pltpu.VMEM_SHARED((N_ACC, F), jnp.float32),
            pltpu.VMEM_SHARED((N_ACC, F), jnp.float32),
            pltpu.SemaphoreType.DMA((NBUF,)),
            pltpu.SemaphoreType.DMA((NBUF,)),
        ],
    )(feat, ei)


@jax.jit
def _segment_sum_sc2(parts, ei, bias):
    return pl.kernel(
        _seg2_body,
        mesh=plsc.VectorSubcoreMesh(**_SC_MESH),
        compiler_params=_SC_PARAMS,
        out_type=jax.ShapeDtypeStruct((NC, N_ACC, F), jnp.float32),
        scratch_types=[
            pltpu.VMEM((MAXCH, CHUNK), jnp.int32),
            pltpu.VMEM((MAXCH, CHUNK), jnp.int32),
            pltpu.VMEM((NBUF, CHUNK, F), jnp.float32),
            pltpu.VMEM((SLAB, F), jnp.float32),
            pltpu.VMEM((SLAB, F), jnp.float32),
            pltpu.VMEM((F,), jnp.float32),
            pltpu.VMEM_SHARED((N_ACC, F), jnp.float32),
            pltpu.VMEM_SHARED((N_ACC, F), jnp.float32),
            pltpu.SemaphoreType.DMA((NBUF,)),
            pltpu.SemaphoreType.DMA((NBUF,)),
        ],
    )(parts, ei, bias)


def _mm_body(x_ref, w_ref, o_ref):
    o_ref[...] = jnp.dot(x_ref[...], w_ref[...],
                         preferred_element_type=jnp.float32)


@jax.jit
def _x_w1(x, W1):
    blk = N // 2
    return pl.pallas_call(
        _mm_body,
        grid=(2,),
        in_specs=[
            pl.BlockSpec((blk, 128), lambda i: (i, 0)),
            pl.BlockSpec((128, F), lambda i: (0, 0)),
        ],
        out_specs=pl.BlockSpec((blk, F), lambda i: (i, 0)),
        out_shape=jax.ShapeDtypeStruct((N, F), jnp.float32),
    )(x, W1)


def _head_body(p_ref, w2_ref, b2_ref, wd_ref, bd_ref, wo_ref, bo_ref,
               o_ref):
    agg = p_ref[0] + p_ref[1]
    h2 = jnp.maximum(
        jnp.dot(agg, w2_ref[...], preferred_element_type=jnp.float32)
        + b2_ref[...], 0.0)
    h3 = jnp.maximum(
        jnp.dot(h2, wd_ref[...], preferred_element_type=jnp.float32)
        + bd_ref[...], 0.0)
    z = jnp.dot(h3, wo_ref[...], preferred_element_type=jnp.float32) \
        + bo_ref[...]
    o_ref[...] = 1.0 / (1.0 + jnp.exp(-z))


@jax.jit
def _head(p, W2, b2, Wd, bd, Wo, bo):
    blk = N // 5
    return pl.pallas_call(
        _head_body,
        grid=(5,),
        in_specs=[
            pl.BlockSpec((2, blk, F), lambda i: (0, i, 0)),
            pl.BlockSpec((F, 64), lambda i: (0, 0)),
            pl.BlockSpec((1, 64), lambda i: (0, 0)),
            pl.BlockSpec((64, 128), lambda i: (0, 0)),
            pl.BlockSpec((1, 128), lambda i: (0, 0)),
            pl.BlockSpec((128, 1), lambda i: (0, 0)),
            pl.BlockSpec((1, 1), lambda i: (0, 0)),
        ],
        out_specs=pl.BlockSpec((blk, 1), lambda i: (i, 0)),
        out_shape=jax.ShapeDtypeStruct((N, 1), jnp.float32),
    )(p, W2, b2.reshape(1, 64), Wd,
      bd.reshape(1, 128), Wo, bo.reshape(1, 1))


def kernel(x, edge_index, W1, b1, W2, b2, Wd, bd, Wo, bo):
    ei = edge_index.astype(jnp.int32).reshape(2, ROWS_TOT, CHUNK)
    ei = lax.optimization_barrier(ei)

    t1 = _x_w1(x, W1)
    p1 = _segment_sum_sc1(t1, ei)
    p2 = _segment_sum_sc2(p1, ei, b1)
    return _head(p2, W2, b2, Wd, bd, Wo, bo)
